# Initial kernel scaffold; baseline (speedup 1.0000x reference)
import jax
import jax.numpy as jnp
from jax.experimental import pallas as pl


def kernel(s, r, o, t, E_im, E_re, R_im, R_re, Rs_im, Rs_re, Ro_im, Ro_re, Ts_im, Ts_re, To_im, To_re):
    raise NotImplementedError("write your pallas kernel here")


# speedup vs baseline: 2.8091x; 2.8091x over previous
import functools

import jax
import jax.numpy as jnp
from jax import lax
from jax.experimental import pallas as pl
from jax.experimental.pallas import tpu as pltpu
from jax.experimental.pallas import tpu_sc as plsc

NC = 2
NS = 16
NW = NC * NS
L = 16

D = 64
CH = 128
NG = D // L


def _sc_score(B, n_chunks):
    mesh = plsc.VectorSubcoreMesh(
        core_axis_name="c", subcore_axis_name="s", num_cores=NC, num_subcores=NS
    )
    q_per_w = B // NW
    assert q_per_w == n_chunks * CH

    row_buf = lambda: pltpu.VMEM((CH, D), jnp.float32)
    idx_buf = lambda: pltpu.VMEM((CH,), jnp.int32)

    @functools.partial(
        pl.kernel,
        out_type=jax.ShapeDtypeStruct((B,), jnp.float32),
        mesh=mesh,
        compiler_params=pltpu.CompilerParams(
            needs_layout_passes=False, use_tc_tiling_on_sc=False
        ),
        scratch_types=dict(
            sidx=idx_buf(), ridx=idx_buf(), oidx=idx_buf(), tidx=idx_buf(),
            sre=row_buf(), sim=row_buf(), ore=row_buf(), oim=row_buf(),
            rre=row_buf(), rim=row_buf(), rsre=row_buf(), rsim=row_buf(),
            rore=row_buf(), roim=row_buf(), tre=row_buf(), tim=row_buf(),
            part=pltpu.VMEM((L, L), jnp.float32),
            res=pltpu.VMEM((CH,), jnp.float32),
            sem=pltpu.SemaphoreType.DMA,
        ),
    )
    def score(s_h, r_h, o_h, t_h, E_im, E_re, R_im, R_re, Rs_im, Rs_re,
              Ro_im, Ro_re, Ts_im, Ts_re, out_h, *, sidx, ridx, oidx, tidx,
              sre, sim, ore, oim, rre, rim, rsre, rsim, rore, roim, tre, tim,
              part, res, sem):
        wid = lax.axis_index("s") * NC + lax.axis_index("c")
        base = wid * q_per_w
        for ch in range(n_chunks):
            cbase = base + ch * CH
            pltpu.sync_copy(s_h.at[pl.ds(cbase, CH)], sidx)
            pltpu.sync_copy(r_h.at[pl.ds(cbase, CH)], ridx)
            pltpu.sync_copy(o_h.at[pl.ds(cbase, CH)], oidx)
            pltpu.sync_copy(t_h.at[pl.ds(cbase, CH)], tidx)
            cps = [
                pltpu.async_copy(E_re.at[sidx], sre, sem),
                pltpu.async_copy(E_im.at[sidx], sim, sem),
                pltpu.async_copy(E_re.at[oidx], ore, sem),
                pltpu.async_copy(E_im.at[oidx], oim, sem),
                pltpu.async_copy(R_re.at[ridx], rre, sem),
                pltpu.async_copy(R_im.at[ridx], rim, sem),
                pltpu.async_copy(Rs_re.at[ridx], rsre, sem),
                pltpu.async_copy(Rs_im.at[ridx], rsim, sem),
                pltpu.async_copy(Ro_re.at[ridx], rore, sem),
                pltpu.async_copy(Ro_im.at[ridx], roim, sem),
                pltpu.async_copy(Ts_re.at[tidx], tre, sem),
                pltpu.async_copy(Ts_im.at[tidx], tim, sem),
            ]
            for cp in cps:
                cp.wait()

            def gbody(qg, carry):
                for j in range(L):
                    q = qg * L + j
                    acc = jnp.zeros((L,), jnp.float32)
                    for g in range(NG):
                        sl = pl.ds(g * L, L)
                        sr = sre[q, sl]
                        si = sim[q, sl]
                        orv = ore[q, sl]
                        oi = oim[q, sl]
                        rr = rre[q, sl]
                        ri = rim[q, sl]
                        rsr = rsre[q, sl]
                        rsi = rsim[q, sl]
                        ror = rore[q, sl]
                        roi = roim[q, sl]
                        tr = tre[q, sl]
                        ti = tim[q, sl]
                        a = sr * rr - si * ri + tr * ror - ti * roi
                        b = sr * ri + si * rr + tr * roi + ti * ror
                        c = sr * rsr - si * rsi
                        d = sr * rsi + si * rsr
                        acc = acc + (a * orv + b * oi + c * tr + d * ti)
                    part[j] = plsc.cumsum(acc)
                rows = lax.iota(jnp.int32, L)
                cols = jnp.full((L,), L - 1, jnp.int32)
                res[pl.ds(qg * L, L)] = plsc.load_gather(part, [rows, cols])
                return carry

            lax.fori_loop(0, CH // L, gbody, 0)
            pltpu.sync_copy(res, out_h.at[pl.ds(cbase, CH)])

    return score


def kernel(s, r, o, t, E_im, E_re, R_im, R_re, Rs_im, Rs_re, Ro_im, Ro_re,
           Ts_im, Ts_re, To_im, To_re):
    del To_im, To_re
    B = s.shape[0]
    s_idx = s.reshape(B).astype(jnp.int32)
    r_idx = r.reshape(B).astype(jnp.int32)
    o_idx = o.reshape(B).astype(jnp.int32)
    t_idx = t[:, 0, 0].astype(jnp.int32)
    score = _sc_score(B, B // (NW * CH))
    out = score(s_idx, r_idx, o_idx, t_idx, E_im, E_re, R_im, R_re,
                Rs_im, Rs_re, Ro_im, Ro_re, Ts_im, Ts_re)
    return out.reshape(B, 1)


# speedup vs baseline: 3.1004x; 1.1037x over previous
import functools

import jax
import jax.numpy as jnp
from jax import lax
from jax.experimental import pallas as pl
from jax.experimental.pallas import tpu as pltpu
from jax.experimental.pallas import tpu_sc as plsc

NC = 2
NS = 16
NW = NC * NS
L = 16

D = 64
CH = 128
NG = D // L


def _sc_score(B, n_chunks):
    mesh = plsc.VectorSubcoreMesh(
        core_axis_name="c", subcore_axis_name="s", num_cores=NC, num_subcores=NS
    )
    q_per_w = B // NW
    assert q_per_w == n_chunks * CH

    idx_buf = lambda: pltpu.VMEM((CH,), jnp.int32)

    @functools.partial(
        pl.kernel,
        out_type=jax.ShapeDtypeStruct((B,), jnp.float32),
        mesh=mesh,
        compiler_params=pltpu.CompilerParams(needs_layout_passes=False),
        scratch_types=dict(
            sidx=idx_buf(), ridx=idx_buf(), oidx=idx_buf(), tidx=idx_buf(),
            sbuf=pltpu.VMEM((CH, 2 * D), jnp.float32),
            obuf=pltpu.VMEM((CH, 2 * D), jnp.float32),
            rbuf=pltpu.VMEM((CH, 6 * D), jnp.float32),
            tbuf=pltpu.VMEM((CH, 2 * D), jnp.float32),
            part=pltpu.VMEM((L, L), jnp.float32),
            res=pltpu.VMEM((CH,), jnp.float32),
            sem=pltpu.SemaphoreType.DMA,
        ),
    )
    def score(s_h, r_h, o_h, t_h, Ecat, Rcat, Tcat, out_h, *, sidx, ridx,
              oidx, tidx, sbuf, obuf, rbuf, tbuf, part, res, sem):
        wid = lax.axis_index("s") * NC + lax.axis_index("c")
        base = wid * q_per_w
        for ch in range(n_chunks):
            cbase = base + ch * CH
            pltpu.sync_copy(s_h.at[pl.ds(cbase, CH)], sidx)
            pltpu.sync_copy(r_h.at[pl.ds(cbase, CH)], ridx)
            pltpu.sync_copy(o_h.at[pl.ds(cbase, CH)], oidx)
            pltpu.sync_copy(t_h.at[pl.ds(cbase, CH)], tidx)
            cps = [
                pltpu.async_copy(Ecat.at[sidx], sbuf, sem),
                pltpu.async_copy(Ecat.at[oidx], obuf, sem),
                pltpu.async_copy(Rcat.at[ridx], rbuf, sem),
                pltpu.async_copy(Tcat.at[tidx], tbuf, sem),
            ]
            for cp in cps:
                cp.wait()

            def gbody(qg, carry):
                for j in range(L):
                    q = qg * L + j
                    acc = jnp.zeros((L,), jnp.float32)
                    for g in range(NG):
                        o0 = g * L
                        sr = sbuf[q, pl.ds(o0, L)]
                        si = sbuf[q, pl.ds(D + o0, L)]
                        orv = obuf[q, pl.ds(o0, L)]
                        oi = obuf[q, pl.ds(D + o0, L)]
                        rr = rbuf[q, pl.ds(o0, L)]
                        ri = rbuf[q, pl.ds(D + o0, L)]
                        rsr = rbuf[q, pl.ds(2 * D + o0, L)]
                        rsi = rbuf[q, pl.ds(3 * D + o0, L)]
                        ror = rbuf[q, pl.ds(4 * D + o0, L)]
                        roi = rbuf[q, pl.ds(5 * D + o0, L)]
                        tr = tbuf[q, pl.ds(o0, L)]
                        ti = tbuf[q, pl.ds(D + o0, L)]
                        a = sr * rr - si * ri + tr * ror - ti * roi
                        b = sr * ri + si * rr + tr * roi + ti * ror
                        c = sr * rsr - si * rsi
                        d = sr * rsi + si * rsr
                        acc = acc + (a * orv + b * oi + c * tr + d * ti)
                    part[j] = plsc.cumsum(acc)
                rows = lax.iota(jnp.int32, L)
                cols = jnp.full((L,), L - 1, jnp.int32)
                res[pl.ds(qg * L, L)] = plsc.load_gather(part, [rows, cols])
                return carry

            lax.fori_loop(0, CH // L, gbody, 0)
            pltpu.sync_copy(res, out_h.at[pl.ds(cbase, CH)])

    return score


def kernel(s, r, o, t, E_im, E_re, R_im, R_re, Rs_im, Rs_re, Ro_im, Ro_re,
           Ts_im, Ts_re, To_im, To_re):
    del To_im, To_re
    B = s.shape[0]
    s_idx = s.reshape(B).astype(jnp.int32)
    r_idx = r.reshape(B).astype(jnp.int32)
    o_idx = o.reshape(B).astype(jnp.int32)
    t_idx = t[:, 0, 0].astype(jnp.int32)
    Ecat = jnp.concatenate([E_re, E_im], axis=1)
    Rcat = jnp.concatenate([R_re, R_im, Rs_re, Rs_im, Ro_re, Ro_im], axis=1)
    Tcat = jnp.concatenate([Ts_re, Ts_im], axis=1)
    score = _sc_score(B, B // (NW * CH))
    out = score(s_idx, r_idx, o_idx, t_idx, Ecat, Rcat, Tcat)
    return out.reshape(B, 1)


# speedup vs baseline: 3.3642x; 1.0851x over previous
import functools

import jax
import jax.numpy as jnp
from jax import lax
from jax.experimental import pallas as pl
from jax.experimental.pallas import tpu as pltpu
from jax.experimental.pallas import tpu_sc as plsc

NC = 2
NS = 16
NW = NC * NS
L = 16

D = 64
CH = 64
NG = D // L


def _sc_score(B, n_chunks):
    mesh = plsc.VectorSubcoreMesh(
        core_axis_name="c", subcore_axis_name="s", num_cores=NC, num_subcores=NS
    )
    q_per_w = B // NW
    assert q_per_w == n_chunks * CH and n_chunks % 2 == 0

    idx_buf = lambda: pltpu.VMEM((CH,), jnp.int32)
    ebuf = lambda: pltpu.VMEM((CH, 2 * D), jnp.float32)

    @functools.partial(
        pl.kernel,
        out_type=jax.ShapeDtypeStruct((B,), jnp.float32),
        mesh=mesh,
        compiler_params=pltpu.CompilerParams(needs_layout_passes=False),
        scratch_types=dict(
            idx=[[idx_buf() for _ in range(4)] for _ in range(2)],
            sbuf=[ebuf() for _ in range(2)],
            obuf=[ebuf() for _ in range(2)],
            rbuf=[pltpu.VMEM((CH, 6 * D), jnp.float32) for _ in range(2)],
            tbuf=[ebuf() for _ in range(2)],
            part=pltpu.VMEM((L, L), jnp.float32),
            res=pltpu.VMEM((CH,), jnp.float32),
            sem=[pltpu.SemaphoreType.DMA for _ in range(2)],
        ),
    )
    def score(idx_h, Ecat, Rcat, Tcat, out_h, *, idx, sbuf, obuf, rbuf, tbuf,
              part, res, sem):
        wid = lax.axis_index("s") * NC + lax.axis_index("c")
        base = wid * q_per_w

        def fetch(ch, slot):
            cbase = base + ch * CH
            for k in range(4):
                pltpu.sync_copy(idx_h.at[pl.ds(k * B + cbase, CH)], idx[slot][k])
            pltpu.async_copy(Ecat.at[idx[slot][0]], sbuf[slot], sem[slot])
            pltpu.async_copy(Rcat.at[idx[slot][1]], rbuf[slot], sem[slot])
            pltpu.async_copy(Ecat.at[idx[slot][2]], obuf[slot], sem[slot])
            pltpu.async_copy(Tcat.at[idx[slot][3]], tbuf[slot], sem[slot])

        def drain(slot):
            pltpu.make_async_copy(Ecat.at[idx[slot][0]], sbuf[slot], sem[slot]).wait()
            pltpu.make_async_copy(Rcat.at[idx[slot][1]], rbuf[slot], sem[slot]).wait()
            pltpu.make_async_copy(Ecat.at[idx[slot][2]], obuf[slot], sem[slot]).wait()
            pltpu.make_async_copy(Tcat.at[idx[slot][3]], tbuf[slot], sem[slot]).wait()

        def compute(slot):
            sb, ob, rb, tb = sbuf[slot], obuf[slot], rbuf[slot], tbuf[slot]

            def gbody(qg, carry):
                for j in range(L):
                    q = qg * L + j
                    acc = jnp.zeros((L,), jnp.float32)
                    for g in range(NG):
                        o0 = g * L
                        sr = sb[q, pl.ds(o0, L)]
                        si = sb[q, pl.ds(D + o0, L)]
                        orv = ob[q, pl.ds(o0, L)]
                        oi = ob[q, pl.ds(D + o0, L)]
                        rr = rb[q, pl.ds(o0, L)]
                        ri = rb[q, pl.ds(D + o0, L)]
                        rsr = rb[q, pl.ds(2 * D + o0, L)]
                        rsi = rb[q, pl.ds(3 * D + o0, L)]
                        ror = rb[q, pl.ds(4 * D + o0, L)]
                        roi = rb[q, pl.ds(5 * D + o0, L)]
                        tr = tb[q, pl.ds(o0, L)]
                        ti = tb[q, pl.ds(D + o0, L)]
                        a = sr * rr - si * ri + tr * ror - ti * roi
                        b = sr * ri + si * rr + tr * roi + ti * ror
                        c = sr * rsr - si * rsi
                        d = sr * rsi + si * rsr
                        acc = acc + (a * orv + b * oi + c * tr + d * ti)
                    part[j] = plsc.cumsum(acc)
                rows = lax.iota(jnp.int32, L)
                cols = jnp.full((L,), L - 1, jnp.int32)
                res[pl.ds(qg * L, L)] = plsc.load_gather(part, [rows, cols])
                return carry

            lax.fori_loop(0, CH // L, gbody, 0)

        npairs = n_chunks // 2
        fetch(0, 0)

        def pair_body(p, carry):
            c0 = 2 * p
            drain(0)
            fetch(c0 + 1, 1)
            compute(0)
            pltpu.sync_copy(res, out_h.at[pl.ds(base + c0 * CH, CH)])
            drain(1)

            @pl.when(p + 1 < npairs)
            def _():
                fetch(c0 + 2, 0)

            compute(1)
            pltpu.sync_copy(res, out_h.at[pl.ds(base + (c0 + 1) * CH, CH)])
            return carry

        lax.fori_loop(0, npairs, pair_body, 0)

    return score


def kernel(s, r, o, t, E_im, E_re, R_im, R_re, Rs_im, Rs_re, Ro_im, Ro_re,
           Ts_im, Ts_re, To_im, To_re):
    del To_im, To_re
    B = s.shape[0]
    idx_h = jnp.concatenate(
        [s.reshape(B), r.reshape(B), o.reshape(B), t[:, 0, 0]]
    ).astype(jnp.int32)
    Ecat = jnp.concatenate([E_re, E_im], axis=1)
    Rcat = jnp.concatenate([R_re, R_im, Rs_re, Rs_im, Ro_re, Ro_im], axis=1)
    Tcat = jnp.concatenate([Ts_re, Ts_im], axis=1)
    score = _sc_score(B, B // (NW * CH))
    out = score(idx_h, Ecat, Rcat, Tcat)
    return out.reshape(B, 1)
